# baseline (speedup 1.0000x reference)
import functools

import jax
import jax.numpy as jnp
from jax import lax
from jax.experimental import pallas as pl
from jax.experimental.pallas import tpu as pltpu
from jax.experimental.pallas import tpu_sc as plsc


def _conv(x, w, b, stride, pad):
    out = lax.conv_general_dilated(
        x, w, (stride, stride), ((pad, pad), (pad, pad)),
        dimension_numbers=('NCHW', 'OIHW', 'NCHW'))
    return out + b.reshape(1, -1, 1, 1)


def _conv_t(x, w, b, stride, pad):
    k = w.shape[2]
    w2 = jnp.flip(w, (2, 3)).transpose(1, 0, 2, 3)
    p = k - 1 - pad
    out = lax.conv_general_dilated(
        x, w2, (1, 1), ((p, p), (p, p)), lhs_dilation=(stride, stride),
        dimension_numbers=('NCHW', 'OIHW', 'NCHW'))
    return out + b.reshape(1, -1, 1, 1)


def _bn(x):
    m = x.mean(axis=(0, 2, 3), keepdims=True)
    v = x.var(axis=(0, 2, 3), keepdims=True)
    return (x - m) / jnp.sqrt(v + 1e-5)


def _vq_argmin_body(z_ref, w_ref, wt_ref, idx_ref, zq_ref):
    zt = z_ref[...]
    K = w_ref.shape[0]
    T = zt.shape[0]
    zw = jnp.dot(zt, wt_ref[...], preferred_element_type=jnp.float32,
                 precision=lax.Precision.HIGHEST)
    zn = jnp.sum(zt * zt, axis=1)
    wn = jnp.sum(wt_ref[...] * wt_ref[...], axis=0)
    d = zn[:, None] - 2.0 * zw + wn[None, :]
    ilane = lax.broadcasted_iota(jnp.int32, (T, K), 1)
    min1 = jnp.min(d, axis=1)
    a1 = jnp.min(jnp.where(d == min1[:, None], ilane, K), axis=1)
    d2m = jnp.where(ilane == a1[:, None], jnp.inf, d)
    min2 = jnp.min(d2m, axis=1)
    a2 = jnp.min(jnp.where(d2m == min2[:, None], ilane, K), axis=1)
    oh1 = (ilane == a1[:, None]).astype(jnp.float32)
    oh2 = (ilane == a2[:, None]).astype(jnp.float32)
    q1 = jnp.dot(oh1, w_ref[...], preferred_element_type=jnp.float32,
                 precision=lax.Precision.HIGHEST)
    q2 = jnp.dot(oh2, w_ref[...], preferred_element_type=jnp.float32,
                 precision=lax.Precision.HIGHEST)
    D = zt.shape[1]
    df1 = (zt - q1).reshape(T // 128, 128, D)
    e1 = jnp.sum(df1 * df1, axis=2).reshape(T)
    df2 = (zt - q2).reshape(T // 128, 128, D)
    e2 = jnp.sum(df2 * df2, axis=2).reshape(T)
    better = (e2 < e1) | ((e2 == e1) & (a2 < a1))
    idx_ref[0, 0] = jnp.where(better, a2, a1)
    q = jnp.where(better[:, None], q2, q1)
    zq_ref[0] = zt + (q - zt)


def _vq_argmin(z2, vq_w):
    NP, D = z2.shape
    T = 1024
    G = NP // T
    K = vq_w.shape[0]
    return pl.pallas_call(
        _vq_argmin_body,
        grid=(G,),
        in_specs=[pl.BlockSpec((T, D), lambda i: (i, 0)),
                  pl.BlockSpec((K, D), lambda i: (0, 0)),
                  pl.BlockSpec((D, K), lambda i: (0, 0))],
        out_specs=[pl.BlockSpec((1, 1, T), lambda i: (i, 0, 0)),
                   pl.BlockSpec((1, T, D), lambda i: (i, 0, 0))],
        out_shape=[jax.ShapeDtypeStruct((G, 1, T), jnp.int32),
                   jax.ShapeDtypeStruct((G, T, D), jnp.float32)],
    )(z2, vq_w, vq_w.T)


def _sc_counts(idx3, n_real, K):
    NW, NCHUNK, CH = idx3.shape
    BPW = NCHUNK * CH
    NC = plsc.get_sparse_core_info().num_cores

    mesh = plsc.VectorSubcoreMesh(core_axis_name="c", subcore_axis_name="s")

    @functools.partial(
        pl.kernel,
        mesh=mesh,
        compiler_params=pltpu.CompilerParams(needs_layout_passes=False),
        out_type=jax.ShapeDtypeStruct((NW, K), jnp.float32),
        scratch_types=[
            pltpu.VMEM((NCHUNK, CH), jnp.int32),
            pltpu.VMEM((K,), jnp.float32),
        ],
    )
    def body(idx_hbm, counts_out, idx_v, cnt_v):
        wid = lax.axis_index("s") * NC + lax.axis_index("c")
        base = wid * BPW
        pltpu.sync_copy(idx_hbm.at[wid], idx_v)
        for i in range(K // 16):
            cnt_v[pl.ds(i * 16, 16)] = jnp.zeros((16,), jnp.float32)
        ones = jnp.ones((16,), jnp.float32)
        for c in range(NCHUNK):
            for j in range(0, CH, 16):
                iv = idx_v[c, pl.ds(j, 16)]
                gpos = base + c * CH + j + lax.iota(jnp.int32, 16)
                plsc.addupdate_scatter(cnt_v, [iv], ones, mask=gpos < n_real)
        pltpu.sync_copy(cnt_v, counts_out.at[wid])

    return body(idx3)


def kernel(inputs, enc0_w, enc0_b, enc1_w, enc1_b, enc2_w, enc2_b, enc3_w,
           enc3_b, enc4_w, enc4_b, res0a_w, res0a_b, res0b_w, res0b_b,
           res1a_w, res1a_b, res1b_w, res1b_b, vq_w, dec0_w, dec0_b, dec1_w,
           dec1_b, dec2_w, dec2_b, dec3_w, dec3_b, channel_var):
    h = _conv(inputs, enc0_w, enc0_b, 1, 0)
    h = _conv(h, enc1_w, enc1_b, 2, 1); h = _bn(h); h = jax.nn.relu(h)
    h = _conv(h, enc2_w, enc2_b, 2, 1); h = _bn(h); h = jax.nn.relu(h)
    h = _conv(h, enc3_w, enc3_b, 2, 1); h = _bn(h); h = jax.nn.relu(h)
    h = _conv(h, enc4_w, enc4_b, 1, 1); h = _bn(h)
    for wa, ba, wb, bb in ((res0a_w, res0a_b, res0b_w, res0b_b),
                           (res1a_w, res1a_b, res1b_w, res1b_b)):
        r = jax.nn.relu(h)
        r = _conv(r, wa, ba, 1, 1); r = _bn(r); r = jax.nn.relu(r)
        r = _conv(r, wb, bb, 1, 0); r = _bn(r)
        h = h + r
    z = h
    Bt, D, hh, ww = z.shape
    K = vq_w.shape[0]
    P = hh * ww
    N = Bt * P

    NW = 32
    CH = 112
    NCHUNK = -(-N // (NW * CH))
    B = NW * NCHUNK * CH
    zf = z.transpose(0, 2, 3, 1).reshape(N, D)
    zf_pad = jnp.concatenate(
        [zf, jnp.zeros((B - N, D), jnp.float32)], axis=0)
    idx3, zq3 = _vq_argmin(zf_pad, vq_w)
    counts_pw = _sc_counts(idx3.reshape(NW, NCHUNK, CH), N, K)
    zq = zq3.reshape(B, D)[:N].reshape(Bt, P, D).transpose(0, 2, 1)
    zq = zq.reshape(Bt, D, hh, ww)
    e_latent = jnp.mean((zq - z) ** 2)
    q_latent = e_latent
    c_loss = q_latent + 0.25 * e_latent
    avg_probs = counts_pw.sum(axis=0) / N
    perplexity = jnp.exp(-jnp.sum(avg_probs * jnp.log(avg_probs + 1e-10)))

    d = _conv_t(zq, dec0_w, dec0_b, 2, 1); d = jax.nn.relu(d)
    d = _conv_t(d, dec1_w, dec1_b, 2, 1); d = jax.nn.relu(d)
    d = _conv_t(d, dec2_w, dec2_b, 2, 1); d = jax.nn.relu(d)
    decoded = _conv(d, dec3_w, dec3_b, 1, 0)
    recon_loss = jnp.mean(((decoded - inputs) ** 2) / channel_var)
    total_loss = recon_loss + c_loss
    return decoded, recon_loss, c_loss, perplexity, total_loss
